# baseline (speedup 1.0000x reference)
import functools

import jax
import jax.numpy as jnp
from jax import lax
from jax.experimental import pallas as pl
from jax.experimental.pallas import tpu as pltpu
from jax.experimental.pallas import tpu_sc as plsc

N = 10000
E = 320000
IN_DIM = 128
HID = 128
P = 8192
L = 20

NC = 2
NS = 16
NW = NC * NS
C = 128
NR = 10112
RPT = NR // NS
EPW_CH = 80
EP = NW * EPW_CH * C
IDXB = 16
NBLK = EPW_CH // IDXB
GPW_CH = (2 * P) // (NW * C)
DUMP = N

M_UPD = E + N
NW2 = 16
XCS = 20640
SYM_CH = 162
CAP2 = SYM_CH * C


@functools.cache
def _sc_kernels():
    mesh = plsc.VectorSubcoreMesh(core_axis_name="c", subcore_axis_name="s",
                                  num_cores=NC, num_subcores=NS)

    @functools.partial(
        pl.kernel,
        out_type=jax.ShapeDtypeStruct((NC, NR, HID), jnp.float32),
        mesh=mesh,
        scratch_types=[
            pltpu.VMEM((IDXB, C), jnp.int32),
            pltpu.VMEM((IDXB, C), jnp.int32),
            pltpu.VMEM((2, C, HID), jnp.float32),
            pltpu.VMEM_SHARED((NR, HID), jnp.float32),
            pltpu.SemaphoreType.DMA((2,)),
        ],
    )
    def edge_scatter(z_hbm, row_hbm, col_hbm, zeros_hbm, out_hbm,
                     rbuf, cbuf, gbufs, acc, sems):
        c = lax.axis_index("c")
        s = lax.axis_index("s")
        wid = s * NC + c

        pltpu.sync_copy(zeros_hbm.at[pl.ds(s * RPT, RPT)],
                        acc.at[pl.ds(s * RPT, RPT)])
        plsc.subcore_barrier()

        def blk(jb, carry):
            pltpu.sync_copy(row_hbm.at[wid, pl.ds(jb * IDXB, IDXB)], rbuf)
            pltpu.sync_copy(col_hbm.at[wid, pl.ds(jb * IDXB, IDXB)], cbuf)
            pltpu.async_copy(z_hbm.at[rbuf.at[0]], gbufs.at[0], sems.at[0])
            for i in range(IDXB):
                slot = i % 2
                if i + 1 < IDXB:
                    pltpu.async_copy(z_hbm.at[rbuf.at[i + 1]],
                                     gbufs.at[1 - slot], sems.at[1 - slot])
                pltpu.make_async_copy(z_hbm.at[rbuf.at[i]], gbufs.at[slot],
                                      sems.at[slot]).wait()
                pltpu.sync_copy(gbufs.at[slot], acc.at[cbuf.at[i]], add=True)
            return carry

        lax.fori_loop(0, NBLK, blk, 0)
        plsc.subcore_barrier()
        pltpu.sync_copy(acc.at[pl.ds(s * RPT, RPT)],
                        out_hbm.at[c, pl.ds(s * RPT, RPT)])

    @functools.partial(
        pl.kernel,
        out_type=jax.ShapeDtypeStruct((2 * P, HID), jnp.float32),
        mesh=mesh,
        scratch_types=[
            pltpu.VMEM((GPW_CH, C), jnp.int32),
            pltpu.VMEM((2, C, HID), jnp.float32),
            pltpu.SemaphoreType.DMA((2,)),
        ],
    )
    def pair_gather(tab_hbm, idx_hbm, out_hbm, ibuf, gbufs, sems):
        c = lax.axis_index("c")
        s = lax.axis_index("s")
        wid = s * NC + c

        pltpu.sync_copy(idx_hbm.at[wid], ibuf)
        pltpu.async_copy(tab_hbm.at[ibuf.at[0]], gbufs.at[0], sems.at[0])
        for i in range(GPW_CH):
            slot = i % 2
            if i + 1 < GPW_CH:
                pltpu.async_copy(tab_hbm.at[ibuf.at[i + 1]],
                                 gbufs.at[1 - slot], sems.at[1 - slot])
            pltpu.make_async_copy(tab_hbm.at[ibuf.at[i]], gbufs.at[slot],
                                  sems.at[slot]).wait()
            base = (wid * GPW_CH + i) * C
            pltpu.sync_copy(gbufs.at[slot], out_hbm.at[pl.ds(base, C)])

    @functools.partial(
        pl.kernel,
        out_type=jax.ShapeDtypeStruct((NC, NR, HID), jnp.float32),
        mesh=mesh,
        scratch_types=[
            pltpu.VMEM((C,), jnp.int32),
            pltpu.VMEM((C,), jnp.int32),
            pltpu.VMEM((C, HID), jnp.float32),
            pltpu.VMEM((C, HID), jnp.float32),
            pltpu.VMEM_SHARED((NR, HID), jnp.float32),
            pltpu.SemaphoreType.DMA,
        ],
    )
    def sym_scatter(hw_hbm, row_hbm, fcol_hbm, nk_hbm, zeros_hbm,
                    out_hbm, rbuf, fcbuf, nkbuf, gbuf, acc, sem):
        c = lax.axis_index("c")
        s = lax.axis_index("s")
        wid = s * NC + c

        pltpu.sync_copy(zeros_hbm.at[pl.ds(s * RPT, RPT)],
                        acc.at[pl.ds(s * RPT, RPT)])
        plsc.subcore_barrier()

        @pl.when(wid < NW2)
        def _active():
            zero8 = tuple(jnp.zeros((16,), jnp.float32) for _ in range(8))

            def chunk(j, acc8):
                pltpu.sync_copy(row_hbm.at[wid, j], rbuf)
                pltpu.sync_copy(fcol_hbm.at[wid, j], fcbuf)
                pltpu.sync_copy(nk_hbm.at[wid, j], nkbuf)
                pltpu.async_copy(hw_hbm.at[rbuf], gbuf, sem).wait()

                def upd(e, a8):
                    nsp = nkbuf[e, pl.ds(0, 16)]
                    ksp = nkbuf[e, pl.ds(64, 16)]
                    out = []
                    for k in range(8):
                        ak = gbuf[e, pl.ds(16 * k, 16)] * nsp + ksp * a8[k]
                        gbuf[e, pl.ds(16 * k, 16)] = ak
                        out.append(ak)
                    return tuple(out)

                acc8 = lax.fori_loop(0, C, upd, acc8)
                pltpu.sync_copy(gbuf, acc.at[fcbuf], add=True)
                return acc8

            lax.fori_loop(0, SYM_CH, chunk, zero8)

        plsc.subcore_barrier()
        pltpu.sync_copy(acc.at[pl.ds(s * RPT, RPT)],
                        out_hbm.at[c, pl.ds(s * RPT, RPT)])

    @functools.partial(
        pl.kernel,
        out_type=jax.ShapeDtypeStruct((NW2 * CAP2, HID), jnp.float32),
        mesh=mesh,
        scratch_types=[
            pltpu.VMEM((C,), jnp.int32),
            pltpu.VMEM((C, HID), jnp.float32),
            pltpu.SemaphoreType.DMA,
        ],
    )
    def sym_gather(tab_hbm, idx_hbm, out_hbm, ibuf, gbuf, sem):
        c = lax.axis_index("c")
        s = lax.axis_index("s")
        wid = s * NC + c

        @pl.when(wid < NW2)
        def _active():
            def chunk(j, carry):
                pltpu.sync_copy(idx_hbm.at[wid, j], ibuf)
                pltpu.async_copy(tab_hbm.at[ibuf], gbuf, sem).wait()
                base = wid * CAP2 + j * C
                pltpu.sync_copy(gbuf, out_hbm.at[pl.ds(base, C)])
                return carry

            lax.fori_loop(0, SYM_CH, chunk, 0)

    return edge_scatter, pair_gather, sym_scatter, sym_gather


RB = 1000
NG = N // RB
PB = 1024
PG = P // PB

_CT1 = (((1,), (1,)), ((), ()))
_CT0 = (((1,), (0,)), ((), ()))


def _enc_body(x_ref, w_ref, h_ref):
    h_ref[...] = jnp.maximum(
        lax.dot_general(x_ref[...], w_ref[...], _CT1,
                        preferred_element_type=jnp.float32), 0.0)


_encode = pl.pallas_call(
    _enc_body,
    grid=(NG,),
    in_specs=[pl.BlockSpec((RB, IN_DIM), lambda i: (i, 0)),
              pl.BlockSpec((HID, IN_DIM), lambda i: (0, 0))],
    out_specs=pl.BlockSpec((RB, HID), lambda i: (i, 0)),
    out_shape=jax.ShapeDtypeStruct((N, HID), jnp.float32),
)


def _dis_body(dp_ref, dis_ref):
    deg = dp_ref[0, :, 0:1] + dp_ref[1, :, 0:1] + 1.0
    dis_ref[...] = jnp.broadcast_to(lax.rsqrt(deg), (NR, HID))


_dis_k = pl.pallas_call(
    _dis_body,
    in_specs=[pl.BlockSpec((NC, NR, HID), lambda: (0, 0, 0))],
    out_specs=pl.BlockSpec((NR, HID), lambda: (0, 0)),
    out_shape=jax.ShapeDtypeStruct((NR, HID), jnp.float32),
)


def _msg_body(h_ref, oa_ref, ws_ref, anti_ref, hw_ref):
    h = h_ref[...]
    anti_ref[...] = -jnp.maximum(
        lax.dot_general(h, oa_ref[...], _CT0,
                        preferred_element_type=jnp.float32), 0.0)
    hw_ref[...] = lax.dot_general(h, ws_ref[...], _CT0,
                                  preferred_element_type=jnp.float32)


_msg = pl.pallas_call(
    _msg_body,
    grid=(NG,),
    in_specs=[pl.BlockSpec((RB, HID), lambda i: (i, 0)),
              pl.BlockSpec((HID, HID), lambda i: (0, 0)),
              pl.BlockSpec((HID, HID), lambda i: (0, 0))],
    out_specs=[pl.BlockSpec((RB, HID), lambda i: (i, 0)),
               pl.BlockSpec((RB, HID), lambda i: (i, 0))],
    out_shape=[jax.ShapeDtypeStruct((N, HID), jnp.float32),
               jax.ShapeDtypeStruct((N, HID), jnp.float32)],
)


def _upd_body(anti_ref, p_ref, h_ref, hn_ref, dh_ref):
    sym = p_ref[0] + p_ref[1]
    dh = jnp.maximum(jnp.tanh(anti_ref[...] + sym), 0.0)
    dh_ref[...] = dh
    hn_ref[...] = h_ref[...] + dh


_upd = pl.pallas_call(
    _upd_body,
    grid=(NG,),
    in_specs=[pl.BlockSpec((RB, HID), lambda i: (i, 0)),
              pl.BlockSpec((NC, RB, HID), lambda i: (0, i, 0)),
              pl.BlockSpec((RB, HID), lambda i: (i, 0))],
    out_specs=[pl.BlockSpec((RB, HID), lambda i: (i, 0)),
               pl.BlockSpec((RB, HID), lambda i: (i, 0))],
    out_shape=[jax.ShapeDtypeStruct((N, HID), jnp.float32),
               jax.ShapeDtypeStruct((N, HID), jnp.float32)],
)


def _gate_body(hs_ref, hd_ref, dhs_ref, dhd_ref, rs_ref, rd_ref, ts_ref,
               w1_ref, w2_ref, g_ref,
               hs_o, hd_o, rs_o, rd_o, ts_o):
    huv = jnp.concatenate([hs_ref[...], hd_ref[...]], axis=1)
    mm = jnp.maximum(
        lax.dot_general(huv, w1_ref[...], _CT1,
                        preferred_element_type=jnp.float32), 0.0)
    a = lax.dot_general(mm, w2_ref[...], _CT1,
                        preferred_element_type=jnp.float32) + g_ref[...]
    tau = jnp.where(a[:, 0:1] >= a[:, 1:2], 1.0, 0.0).astype(jnp.float32)
    dhs = dhs_ref[...]
    dhd = dhd_ref[...]
    hs_o[...] = hs_ref[...] + dhs
    hd_o[...] = hd_ref[...] + dhd
    rs_o[...] = rs_ref[...] + tau * dhs
    rd_o[...] = rd_ref[...] + tau * dhd
    ts_o[...] = ts_ref[...] + tau


_gate = pl.pallas_call(
    _gate_body,
    grid=(PG,),
    in_specs=[pl.BlockSpec((PB, HID), lambda i: (i, 0)),
              pl.BlockSpec((PB, HID), lambda i: (i, 0)),
              pl.BlockSpec((PB, HID), lambda i: (i, 0)),
              pl.BlockSpec((PB, HID), lambda i: (i, 0)),
              pl.BlockSpec((PB, HID), lambda i: (i, 0)),
              pl.BlockSpec((PB, HID), lambda i: (i, 0)),
              pl.BlockSpec((PB, 1), lambda i: (i, 0)),
              pl.BlockSpec((2 * HID, 2 * HID), lambda i: (0, 0)),
              pl.BlockSpec((2, 2 * HID), lambda i: (0, 0)),
              pl.BlockSpec((PB, 2), lambda i: (i, 0))],
    out_specs=[pl.BlockSpec((PB, HID), lambda i: (i, 0)),
               pl.BlockSpec((PB, HID), lambda i: (i, 0)),
               pl.BlockSpec((PB, HID), lambda i: (i, 0)),
               pl.BlockSpec((PB, HID), lambda i: (i, 0)),
               pl.BlockSpec((PB, 1), lambda i: (i, 0))],
    out_shape=[jax.ShapeDtypeStruct((P, HID), jnp.float32),
               jax.ShapeDtypeStruct((P, HID), jnp.float32),
               jax.ShapeDtypeStruct((P, HID), jnp.float32),
               jax.ShapeDtypeStruct((P, HID), jnp.float32),
               jax.ShapeDtypeStruct((P, 1), jnp.float32)],
)


def _score_body(rs_ref, rd_ref, p1_ref, p2_ref, sc_o):
    r = jnp.concatenate([rs_ref[...], rd_ref[...]], axis=1)
    q = jnp.maximum(
        lax.dot_general(r, p1_ref[...], _CT1,
                        preferred_element_type=jnp.float32), 0.0)
    sc_o[...] = lax.dot_general(q, p2_ref[...], _CT1,
                                preferred_element_type=jnp.float32)


_score = pl.pallas_call(
    _score_body,
    grid=(PG,),
    in_specs=[pl.BlockSpec((PB, HID), lambda i: (i, 0)),
              pl.BlockSpec((PB, HID), lambda i: (i, 0)),
              pl.BlockSpec((HID, 2 * HID), lambda i: (0, 0)),
              pl.BlockSpec((1, HID), lambda i: (0, 0))],
    out_specs=pl.BlockSpec((PB, 1), lambda i: (i, 0)),
    out_shape=jax.ShapeDtypeStruct((P, 1), jnp.float32),
)


def kernel(x, edge_index, pairs, W_enc, Omega, W_s_raw, W_fc1, W_fc2, W_fv,
           W_p1, W_p2):
    del W_fv
    Omega_as = Omega - Omega.T
    W_s = (W_s_raw + W_s_raw.T) / 2.0
    idt = edge_index.dtype

    pad = EP - E
    rows_p = jnp.concatenate(
        [edge_index[0], jnp.zeros((pad,), idt)]).reshape(NW, EPW_CH, C)
    cols_p = jnp.concatenate(
        [edge_index[1], jnp.full((pad,), DUMP, idt)]).reshape(NW, EPW_CH, C)
    idx_pairs = jnp.concatenate(
        [pairs[:, 0], pairs[:, 1]]).reshape(NW, GPW_CH, C)

    sl = jnp.arange(N, dtype=idt)
    rows_all = jnp.concatenate([edge_index[0], sl])
    cols_all = jnp.concatenate([edge_index[1], sl])
    order = jnp.argsort(cols_all, stable=True)
    sr = rows_all[order]
    sc_ = cols_all[order]

    gpos = (jnp.arange(NW2) * XCS)[:, None] + jnp.arange(CAP2)[None, :]
    local = jnp.broadcast_to(jnp.arange(CAP2)[None, :], (NW2, CAP2))
    valid = (gpos < M_UPD) & (local < XCS)
    gclip = jnp.minimum(gpos, M_UPD - 1)
    rows_m = jnp.where(valid, sr[gclip], 0).astype(jnp.int32)
    cols_m = jnp.where(valid, sc_[gclip], DUMP).astype(jnp.int32)
    prevc = jnp.concatenate(
        [jnp.full((NW2, 1), -1, jnp.int32), cols_m[:, :-1]], axis=1)
    keep_m = ((cols_m == prevc) & valid).astype(jnp.float32)
    nxtc = jnp.concatenate(
        [cols_m[:, 1:], jnp.full((NW2, 1), -2, jnp.int32)], axis=1)
    fcol_m = jnp.where((cols_m != nxtc) & valid, cols_m, DUMP)
    rows_m3 = rows_m.reshape(NW2, SYM_CH, C)
    cols_m3 = cols_m.reshape(NW2, SYM_CH, C)
    keep_m3 = keep_m.reshape(NW2, SYM_CH, C)
    fcol_m3 = fcol_m.reshape(NW2, SYM_CH, C)

    zeros_nodes = jnp.zeros((NR, HID), jnp.float32)
    ones_nodes = jnp.ones((N, HID), jnp.float32)

    gs = [jax.random.gumbel(jax.random.fold_in(jax.random.key(42), l),
                            (P, 2), jnp.float32) for l in range(L)]

    edge_scatter, pair_gather, sym_scatter, sym_gather = _sc_kernels()

    H = _encode(x, W_enc)
    degp = edge_scatter(ones_nodes, rows_p, cols_p, zeros_nodes)
    dis128 = _dis_k(degp)
    d_a = sym_gather(dis128, rows_m3)
    d_b = sym_gather(dis128, cols_m3)
    normrows = jnp.where(valid.reshape(-1, 1), (d_a * d_b)[:, :64], 0.0)
    keeprows = jnp.broadcast_to(
        keep_m.reshape(NW2 * CAP2, 1), (NW2 * CAP2, 64))
    nk = jnp.concatenate([normrows, keeprows], axis=1).reshape(
        NW2, SYM_CH, C, HID)

    huv0 = pair_gather(H, idx_pairs)
    hs = huv0[:P]
    hd = huv0[P:]
    rs = hs
    rd = hd
    ts = jnp.zeros((P, 1), jnp.float32)

    for l in range(L):
        anti, HW = _msg(H, Omega_as, W_s)
        pparts = sym_scatter(HW, rows_m3, fcol_m3, nk, zeros_nodes)
        H, dH = _upd(anti, pparts, H)
        dhuv = pair_gather(dH, idx_pairs)
        hs, hd, rs, rd, ts = _gate(hs, hd, dhuv[:P], dhuv[P:], rs, rd, ts,
                                   W_fc1, W_fc2, gs[l])

    scores = _score(rs, rd, W_p1, W_p2)
    return (scores[:, 0], ts[:, 0])


# speedup vs baseline: 1.1298x; 1.1298x over previous
import functools

import jax
import jax.numpy as jnp
from jax import lax
from jax.experimental import pallas as pl
from jax.experimental.pallas import tpu as pltpu
from jax.experimental.pallas import tpu_sc as plsc

N = 10000
E = 320000
IN_DIM = 128
HID = 128
P = 8192
L = 20

NC = 2
NS = 16
NW = NC * NS
C = 128
NR = 10112
RPT = NR // NS
EPW_CH = 80
EP = NW * EPW_CH * C
IDXB = 16
NBLK = EPW_CH // IDXB
GPW_CH = (2 * P) // (NW * C)
DUMP = N

M_UPD = E + N
NW2 = 16
XCS = 20640
SYM_CH = 162
CAP2 = SYM_CH * C


@functools.cache
def _sc_kernels():
    mesh = plsc.VectorSubcoreMesh(core_axis_name="c", subcore_axis_name="s",
                                  num_cores=NC, num_subcores=NS)

    @functools.partial(
        pl.kernel,
        out_type=jax.ShapeDtypeStruct((NC, NR, HID), jnp.float32),
        mesh=mesh,
        scratch_types=[
            pltpu.VMEM((IDXB, C), jnp.int32),
            pltpu.VMEM((IDXB, C), jnp.int32),
            pltpu.VMEM((2, C, HID), jnp.float32),
            pltpu.VMEM_SHARED((NR, HID), jnp.float32),
            pltpu.SemaphoreType.DMA((2,)),
        ],
    )
    def edge_scatter(z_hbm, row_hbm, col_hbm, zeros_hbm, out_hbm,
                     rbuf, cbuf, gbufs, acc, sems):
        c = lax.axis_index("c")
        s = lax.axis_index("s")
        wid = s * NC + c

        pltpu.sync_copy(zeros_hbm.at[pl.ds(s * RPT, RPT)],
                        acc.at[pl.ds(s * RPT, RPT)])
        plsc.subcore_barrier()

        def blk(jb, carry):
            pltpu.sync_copy(row_hbm.at[wid, pl.ds(jb * IDXB, IDXB)], rbuf)
            pltpu.sync_copy(col_hbm.at[wid, pl.ds(jb * IDXB, IDXB)], cbuf)
            pltpu.async_copy(z_hbm.at[rbuf.at[0]], gbufs.at[0], sems.at[0])
            for i in range(IDXB):
                slot = i % 2
                if i + 1 < IDXB:
                    pltpu.async_copy(z_hbm.at[rbuf.at[i + 1]],
                                     gbufs.at[1 - slot], sems.at[1 - slot])
                pltpu.make_async_copy(z_hbm.at[rbuf.at[i]], gbufs.at[slot],
                                      sems.at[slot]).wait()
                pltpu.sync_copy(gbufs.at[slot], acc.at[cbuf.at[i]], add=True)
            return carry

        lax.fori_loop(0, NBLK, blk, 0)
        plsc.subcore_barrier()
        pltpu.sync_copy(acc.at[pl.ds(s * RPT, RPT)],
                        out_hbm.at[c, pl.ds(s * RPT, RPT)])

    @functools.partial(
        pl.kernel,
        out_type=jax.ShapeDtypeStruct((2 * P, HID), jnp.float32),
        mesh=mesh,
        scratch_types=[
            pltpu.VMEM((GPW_CH, C), jnp.int32),
            pltpu.VMEM((2, C, HID), jnp.float32),
            pltpu.SemaphoreType.DMA((2,)),
        ],
    )
    def pair_gather(tab_hbm, idx_hbm, out_hbm, ibuf, gbufs, sems):
        c = lax.axis_index("c")
        s = lax.axis_index("s")
        wid = s * NC + c

        pltpu.sync_copy(idx_hbm.at[wid], ibuf)
        pltpu.async_copy(tab_hbm.at[ibuf.at[0]], gbufs.at[0], sems.at[0])
        for i in range(GPW_CH):
            slot = i % 2
            if i + 1 < GPW_CH:
                pltpu.async_copy(tab_hbm.at[ibuf.at[i + 1]],
                                 gbufs.at[1 - slot], sems.at[1 - slot])
            pltpu.make_async_copy(tab_hbm.at[ibuf.at[i]], gbufs.at[slot],
                                  sems.at[slot]).wait()
            base = (wid * GPW_CH + i) * C
            pltpu.sync_copy(gbufs.at[slot], out_hbm.at[pl.ds(base, C)])

    @functools.partial(
        pl.kernel,
        out_type=jax.ShapeDtypeStruct((NC, NR, HID), jnp.float32),
        mesh=mesh,
        scratch_types=[
            pltpu.VMEM((2, C), jnp.int32),
            pltpu.VMEM((2, C), jnp.int32),
            pltpu.VMEM((2, C // 4, HID), jnp.float32),
            pltpu.VMEM((2, C, HID), jnp.float32),
            pltpu.VMEM_SHARED((NR, HID), jnp.float32),
            pltpu.SemaphoreType.DMA((2,)),
        ],
    )
    def sym_scatter(hw_hbm, row_hbm, fcol_hbm, nk_hbm, zeros_hbm,
                    out_hbm, rbufs, fcbufs, nkbufs, gbufs, acc, sems):
        c = lax.axis_index("c")
        s = lax.axis_index("s")
        wid = s * NC + c

        pltpu.sync_copy(zeros_hbm.at[pl.ds(s * RPT, RPT)],
                        acc.at[pl.ds(s * RPT, RPT)])
        plsc.subcore_barrier()

        @pl.when(wid < NW2)
        def _active():
            zero8 = tuple(jnp.zeros((16,), jnp.float32) for _ in range(8))

            def load_chunk(j, slot):
                pltpu.sync_copy(row_hbm.at[wid, j], rbufs.at[slot])
                pltpu.sync_copy(fcol_hbm.at[wid, j], fcbufs.at[slot])
                pltpu.sync_copy(nk_hbm.at[wid, j], nkbufs.at[slot])
                pltpu.async_copy(hw_hbm.at[rbufs.at[slot]], gbufs.at[slot],
                                 sems.at[slot])

            load_chunk(0, 0)

            def pair(jj, acc8):
                for b in range(2):
                    j = 2 * jj + b

                    @pl.when(j + 1 < SYM_CH)
                    def _prefetch():
                        load_chunk(j + 1, 1 - b)

                    pltpu.make_async_copy(hw_hbm.at[rbufs.at[b]],
                                          gbufs.at[b], sems.at[b]).wait()
                    gbuf = gbufs.at[b]
                    nkbuf = nkbufs.at[b]

                    def upd(r, a8):
                        for q in range(4):
                            e = 4 * r + q
                            nsp = nkbuf[r, pl.ds(32 * q, 16)]
                            ksp = nkbuf[r, pl.ds(32 * q + 16, 16)]
                            a8 = tuple(
                                gbuf[e, pl.ds(16 * k, 16)] * nsp
                                + ksp * a8[k]
                                for k in range(8))
                            for k in range(8):
                                gbuf[e, pl.ds(16 * k, 16)] = a8[k]
                        return a8

                    acc8 = lax.fori_loop(0, C // 4, upd, acc8)
                    pltpu.sync_copy(gbufs.at[b], acc.at[fcbufs.at[b]],
                                    add=True)
                return acc8

            lax.fori_loop(0, SYM_CH // 2, pair, zero8)

        plsc.subcore_barrier()
        pltpu.sync_copy(acc.at[pl.ds(s * RPT, RPT)],
                        out_hbm.at[c, pl.ds(s * RPT, RPT)])

    @functools.partial(
        pl.kernel,
        out_type=jax.ShapeDtypeStruct((NW2 * CAP2, HID), jnp.float32),
        mesh=mesh,
        scratch_types=[
            pltpu.VMEM((C,), jnp.int32),
            pltpu.VMEM((C, HID), jnp.float32),
            pltpu.SemaphoreType.DMA,
        ],
    )
    def sym_gather(tab_hbm, idx_hbm, out_hbm, ibuf, gbuf, sem):
        c = lax.axis_index("c")
        s = lax.axis_index("s")
        wid = s * NC + c

        @pl.when(wid < NW2)
        def _active():
            def chunk(j, carry):
                pltpu.sync_copy(idx_hbm.at[wid, j], ibuf)
                pltpu.async_copy(tab_hbm.at[ibuf], gbuf, sem).wait()
                base = wid * CAP2 + j * C
                pltpu.sync_copy(gbuf, out_hbm.at[pl.ds(base, C)])
                return carry

            lax.fori_loop(0, SYM_CH, chunk, 0)

    return edge_scatter, pair_gather, sym_scatter, sym_gather


RB = 1000
NG = N // RB
PB = 1024
PG = P // PB

_CT1 = (((1,), (1,)), ((), ()))
_CT0 = (((1,), (0,)), ((), ()))


def _enc_body(x_ref, w_ref, h_ref):
    h_ref[...] = jnp.maximum(
        lax.dot_general(x_ref[...], w_ref[...], _CT1,
                        preferred_element_type=jnp.float32), 0.0)


_encode = pl.pallas_call(
    _enc_body,
    grid=(NG,),
    in_specs=[pl.BlockSpec((RB, IN_DIM), lambda i: (i, 0)),
              pl.BlockSpec((HID, IN_DIM), lambda i: (0, 0))],
    out_specs=pl.BlockSpec((RB, HID), lambda i: (i, 0)),
    out_shape=jax.ShapeDtypeStruct((N, HID), jnp.float32),
)


def _dis_body(dp_ref, dis_ref):
    deg = dp_ref[0, :, 0:1] + dp_ref[1, :, 0:1] + 1.0
    dis_ref[...] = jnp.broadcast_to(lax.rsqrt(deg), (NR, HID))


_dis_k = pl.pallas_call(
    _dis_body,
    in_specs=[pl.BlockSpec((NC, NR, HID), lambda: (0, 0, 0))],
    out_specs=pl.BlockSpec((NR, HID), lambda: (0, 0)),
    out_shape=jax.ShapeDtypeStruct((NR, HID), jnp.float32),
)


def _msg_body(h_ref, oa_ref, ws_ref, anti_ref, hw_ref):
    h = h_ref[...]
    anti_ref[...] = -jnp.maximum(
        lax.dot_general(h, oa_ref[...], _CT0,
                        preferred_element_type=jnp.float32), 0.0)
    hw_ref[...] = lax.dot_general(h, ws_ref[...], _CT0,
                                  preferred_element_type=jnp.float32)


_msg = pl.pallas_call(
    _msg_body,
    grid=(NG,),
    in_specs=[pl.BlockSpec((RB, HID), lambda i: (i, 0)),
              pl.BlockSpec((HID, HID), lambda i: (0, 0)),
              pl.BlockSpec((HID, HID), lambda i: (0, 0))],
    out_specs=[pl.BlockSpec((RB, HID), lambda i: (i, 0)),
               pl.BlockSpec((RB, HID), lambda i: (i, 0))],
    out_shape=[jax.ShapeDtypeStruct((N, HID), jnp.float32),
               jax.ShapeDtypeStruct((N, HID), jnp.float32)],
)


def _upd_body(anti_ref, p_ref, h_ref, hn_ref, dh_ref):
    sym = p_ref[0] + p_ref[1]
    dh = jnp.maximum(jnp.tanh(anti_ref[...] + sym), 0.0)
    dh_ref[...] = dh
    hn_ref[...] = h_ref[...] + dh


_upd = pl.pallas_call(
    _upd_body,
    grid=(NG,),
    in_specs=[pl.BlockSpec((RB, HID), lambda i: (i, 0)),
              pl.BlockSpec((NC, RB, HID), lambda i: (0, i, 0)),
              pl.BlockSpec((RB, HID), lambda i: (i, 0))],
    out_specs=[pl.BlockSpec((RB, HID), lambda i: (i, 0)),
               pl.BlockSpec((RB, HID), lambda i: (i, 0))],
    out_shape=[jax.ShapeDtypeStruct((N, HID), jnp.float32),
               jax.ShapeDtypeStruct((N, HID), jnp.float32)],
)


def _gate_body(hs_ref, hd_ref, dhs_ref, dhd_ref, rs_ref, rd_ref, ts_ref,
               w1_ref, w2_ref, g_ref,
               hs_o, hd_o, rs_o, rd_o, ts_o):
    huv = jnp.concatenate([hs_ref[...], hd_ref[...]], axis=1)
    mm = jnp.maximum(
        lax.dot_general(huv, w1_ref[...], _CT1,
                        preferred_element_type=jnp.float32), 0.0)
    a = lax.dot_general(mm, w2_ref[...], _CT1,
                        preferred_element_type=jnp.float32) + g_ref[...]
    tau = jnp.where(a[:, 0:1] >= a[:, 1:2], 1.0, 0.0).astype(jnp.float32)
    dhs = dhs_ref[...]
    dhd = dhd_ref[...]
    hs_o[...] = hs_ref[...] + dhs
    hd_o[...] = hd_ref[...] + dhd
    rs_o[...] = rs_ref[...] + tau * dhs
    rd_o[...] = rd_ref[...] + tau * dhd
    ts_o[...] = ts_ref[...] + tau


_gate = pl.pallas_call(
    _gate_body,
    grid=(PG,),
    in_specs=[pl.BlockSpec((PB, HID), lambda i: (i, 0)),
              pl.BlockSpec((PB, HID), lambda i: (i, 0)),
              pl.BlockSpec((PB, HID), lambda i: (i, 0)),
              pl.BlockSpec((PB, HID), lambda i: (i, 0)),
              pl.BlockSpec((PB, HID), lambda i: (i, 0)),
              pl.BlockSpec((PB, HID), lambda i: (i, 0)),
              pl.BlockSpec((PB, 1), lambda i: (i, 0)),
              pl.BlockSpec((2 * HID, 2 * HID), lambda i: (0, 0)),
              pl.BlockSpec((2, 2 * HID), lambda i: (0, 0)),
              pl.BlockSpec((PB, 2), lambda i: (i, 0))],
    out_specs=[pl.BlockSpec((PB, HID), lambda i: (i, 0)),
               pl.BlockSpec((PB, HID), lambda i: (i, 0)),
               pl.BlockSpec((PB, HID), lambda i: (i, 0)),
               pl.BlockSpec((PB, HID), lambda i: (i, 0)),
               pl.BlockSpec((PB, 1), lambda i: (i, 0))],
    out_shape=[jax.ShapeDtypeStruct((P, HID), jnp.float32),
               jax.ShapeDtypeStruct((P, HID), jnp.float32),
               jax.ShapeDtypeStruct((P, HID), jnp.float32),
               jax.ShapeDtypeStruct((P, HID), jnp.float32),
               jax.ShapeDtypeStruct((P, 1), jnp.float32)],
)


def _score_body(rs_ref, rd_ref, p1_ref, p2_ref, sc_o):
    r = jnp.concatenate([rs_ref[...], rd_ref[...]], axis=1)
    q = jnp.maximum(
        lax.dot_general(r, p1_ref[...], _CT1,
                        preferred_element_type=jnp.float32), 0.0)
    sc_o[...] = lax.dot_general(q, p2_ref[...], _CT1,
                                preferred_element_type=jnp.float32)


_score = pl.pallas_call(
    _score_body,
    grid=(PG,),
    in_specs=[pl.BlockSpec((PB, HID), lambda i: (i, 0)),
              pl.BlockSpec((PB, HID), lambda i: (i, 0)),
              pl.BlockSpec((HID, 2 * HID), lambda i: (0, 0)),
              pl.BlockSpec((1, HID), lambda i: (0, 0))],
    out_specs=pl.BlockSpec((PB, 1), lambda i: (i, 0)),
    out_shape=jax.ShapeDtypeStruct((P, 1), jnp.float32),
)


def kernel(x, edge_index, pairs, W_enc, Omega, W_s_raw, W_fc1, W_fc2, W_fv,
           W_p1, W_p2):
    del W_fv
    Omega_as = Omega - Omega.T
    W_s = (W_s_raw + W_s_raw.T) / 2.0
    idt = edge_index.dtype

    pad = EP - E
    rows_p = jnp.concatenate(
        [edge_index[0], jnp.zeros((pad,), idt)]).reshape(NW, EPW_CH, C)
    cols_p = jnp.concatenate(
        [edge_index[1], jnp.full((pad,), DUMP, idt)]).reshape(NW, EPW_CH, C)
    idx_pairs = jnp.concatenate(
        [pairs[:, 0], pairs[:, 1]]).reshape(NW, GPW_CH, C)

    sl = jnp.arange(N, dtype=idt)
    rows_all = jnp.concatenate([edge_index[0], sl])
    cols_all = jnp.concatenate([edge_index[1], sl])
    order = jnp.argsort(cols_all, stable=True)
    sr = rows_all[order]
    sc_ = cols_all[order]

    gpos = (jnp.arange(NW2) * XCS)[:, None] + jnp.arange(CAP2)[None, :]
    local = jnp.broadcast_to(jnp.arange(CAP2)[None, :], (NW2, CAP2))
    valid = (gpos < M_UPD) & (local < XCS)
    gclip = jnp.minimum(gpos, M_UPD - 1)
    rows_m = jnp.where(valid, sr[gclip], 0).astype(jnp.int32)
    cols_m = jnp.where(valid, sc_[gclip], DUMP).astype(jnp.int32)
    prevc = jnp.concatenate(
        [jnp.full((NW2, 1), -1, jnp.int32), cols_m[:, :-1]], axis=1)
    keep_m = ((cols_m == prevc) & valid).astype(jnp.float32)
    nxtc = jnp.concatenate(
        [cols_m[:, 1:], jnp.full((NW2, 1), -2, jnp.int32)], axis=1)
    fcol_m = jnp.where((cols_m != nxtc) & valid, cols_m, DUMP)
    rows_m3 = rows_m.reshape(NW2, SYM_CH, C)
    cols_m3 = cols_m.reshape(NW2, SYM_CH, C)
    keep_m3 = keep_m.reshape(NW2, SYM_CH, C)
    fcol_m3 = fcol_m.reshape(NW2, SYM_CH, C)

    zeros_nodes = jnp.zeros((NR, HID), jnp.float32)
    ones_nodes = jnp.ones((N, HID), jnp.float32)

    gs = [jax.random.gumbel(jax.random.fold_in(jax.random.key(42), l),
                            (P, 2), jnp.float32) for l in range(L)]

    edge_scatter, pair_gather, sym_scatter, sym_gather = _sc_kernels()

    H = _encode(x, W_enc)
    degp = edge_scatter(ones_nodes, rows_p, cols_p, zeros_nodes)
    dis128 = _dis_k(degp)
    d_a = sym_gather(dis128, rows_m3)
    d_b = sym_gather(dis128, cols_m3)
    norm_flat = jnp.where(valid.reshape(-1), d_a[:, 0] * d_b[:, 0], 0.0)
    nk_pairs = jnp.stack([norm_flat, keep_m.reshape(-1)], axis=1)
    nk = jnp.broadcast_to(
        nk_pairs[:, :, None], (NW2 * CAP2, 2, 16)).reshape(
            NW2, SYM_CH, C // 4, HID)

    huv0 = pair_gather(H, idx_pairs)
    hs = huv0[:P]
    hd = huv0[P:]
    rs = hs
    rd = hd
    ts = jnp.zeros((P, 1), jnp.float32)

    for l in range(L):
        anti, HW = _msg(H, Omega_as, W_s)
        pparts = sym_scatter(HW, rows_m3, fcol_m3, nk, zeros_nodes)
        H, dH = _upd(anti, pparts, H)
        dhuv = pair_gather(dH, idx_pairs)
        hs, hd, rs, rd, ts = _gate(hs, hd, dhuv[:P], dhuv[P:], rs, rd, ts,
                                   W_fc1, W_fc2, gs[l])

    scores = _score(rs, rd, W_p1, W_p2)
    return (scores[:, 0], ts[:, 0])


# speedup vs baseline: 1.4511x; 1.2844x over previous
import functools

import jax
import jax.numpy as jnp
from jax import lax
from jax.experimental import pallas as pl
from jax.experimental.pallas import tpu as pltpu
from jax.experimental.pallas import tpu_sc as plsc

N = 10000
E = 320000
IN_DIM = 128
HID = 128
P = 8192
L = 20

NC = 2
NS = 16
NW = NC * NS
C = 128
NR = 10112
RPT = NR // NS
EPW_CH = 80
EP = NW * EPW_CH * C
IDXB = 16
NBLK = EPW_CH // IDXB
GPW_CH = (2 * P) // (NW * C)
DUMP = N

M_UPD = E + N
NW2 = 16
XCS = 20640
SYM_CH = 162
CAP2 = SYM_CH * C


@functools.cache
def _sc_kernels():
    mesh = plsc.VectorSubcoreMesh(core_axis_name="c", subcore_axis_name="s",
                                  num_cores=NC, num_subcores=NS)

    @functools.partial(
        pl.kernel,
        out_type=jax.ShapeDtypeStruct((NC, NR, HID), jnp.float32),
        mesh=mesh,
        scratch_types=[
            pltpu.VMEM((IDXB, C), jnp.int32),
            pltpu.VMEM((IDXB, C), jnp.int32),
            pltpu.VMEM((2, C, HID), jnp.float32),
            pltpu.VMEM_SHARED((NR, HID), jnp.float32),
            pltpu.SemaphoreType.DMA((2,)),
        ],
    )
    def edge_scatter(z_hbm, row_hbm, col_hbm, zeros_hbm, out_hbm,
                     rbuf, cbuf, gbufs, acc, sems):
        c = lax.axis_index("c")
        s = lax.axis_index("s")
        wid = s * NC + c

        pltpu.sync_copy(zeros_hbm.at[pl.ds(s * RPT, RPT)],
                        acc.at[pl.ds(s * RPT, RPT)])
        plsc.subcore_barrier()

        def blk(jb, carry):
            pltpu.sync_copy(row_hbm.at[wid, pl.ds(jb * IDXB, IDXB)], rbuf)
            pltpu.sync_copy(col_hbm.at[wid, pl.ds(jb * IDXB, IDXB)], cbuf)
            pltpu.async_copy(z_hbm.at[rbuf.at[0]], gbufs.at[0], sems.at[0])
            for i in range(IDXB):
                slot = i % 2
                if i + 1 < IDXB:
                    pltpu.async_copy(z_hbm.at[rbuf.at[i + 1]],
                                     gbufs.at[1 - slot], sems.at[1 - slot])
                pltpu.make_async_copy(z_hbm.at[rbuf.at[i]], gbufs.at[slot],
                                      sems.at[slot]).wait()
                pltpu.sync_copy(gbufs.at[slot], acc.at[cbuf.at[i]], add=True)
            return carry

        lax.fori_loop(0, NBLK, blk, 0)
        plsc.subcore_barrier()
        pltpu.sync_copy(acc.at[pl.ds(s * RPT, RPT)],
                        out_hbm.at[c, pl.ds(s * RPT, RPT)])

    @functools.partial(
        pl.kernel,
        out_type=jax.ShapeDtypeStruct((2 * P, HID), jnp.float32),
        mesh=mesh,
        scratch_types=[
            pltpu.VMEM((GPW_CH, C), jnp.int32),
            pltpu.VMEM((2, C, HID), jnp.float32),
            pltpu.SemaphoreType.DMA((2,)),
        ],
    )
    def pair_gather(tab_hbm, idx_hbm, out_hbm, ibuf, gbufs, sems):
        c = lax.axis_index("c")
        s = lax.axis_index("s")
        wid = s * NC + c

        pltpu.sync_copy(idx_hbm.at[wid], ibuf)
        pltpu.async_copy(tab_hbm.at[ibuf.at[0]], gbufs.at[0], sems.at[0])
        for i in range(GPW_CH):
            slot = i % 2
            if i + 1 < GPW_CH:
                pltpu.async_copy(tab_hbm.at[ibuf.at[i + 1]],
                                 gbufs.at[1 - slot], sems.at[1 - slot])
            pltpu.make_async_copy(tab_hbm.at[ibuf.at[i]], gbufs.at[slot],
                                  sems.at[slot]).wait()
            base = (wid * GPW_CH + i) * C
            pltpu.sync_copy(gbufs.at[slot], out_hbm.at[pl.ds(base, C)])

    @functools.partial(
        pl.kernel,
        out_type=jax.ShapeDtypeStruct((NC, NR, HID), jnp.float32),
        mesh=mesh,
        scratch_types=[
            pltpu.VMEM((2, 24, C), jnp.float32),
            pltpu.VMEM((2, 2, C), jnp.int32),
            pltpu.VMEM((2, C, HID), jnp.float32),
            pltpu.VMEM_SHARED((NR, HID), jnp.float32),
            pltpu.SemaphoreType.DMA((2,)),
        ],
    )
    def sym_scatter(hw_hbm, cmb_hbm, zeros_hbm,
                    out_hbm, cmbufs, ibufs, gbufs, acc, sems):
        c = lax.axis_index("c")
        s = lax.axis_index("s")
        wid = s * NC + c

        pltpu.sync_copy(zeros_hbm.at[pl.ds(s * RPT, RPT)],
                        acc.at[pl.ds(s * RPT, RPT)])
        plsc.subcore_barrier()

        @pl.when(wid < NW2)
        def _active():
            zero8 = tuple(jnp.zeros((16,), jnp.float32) for _ in range(8))
            one = jnp.ones((16,), jnp.float32)
            zero = jnp.zeros((16,), jnp.float32)

            def load_chunk(j, slot):
                pltpu.sync_copy(cmb_hbm.at[wid, j], cmbufs.at[slot])
                for g in range(8):
                    sli = pl.ds(16 * g, 16)
                    ibufs[slot, 0, sli] = cmbufs[slot, 16, sli].astype(
                        jnp.int32)
                    ibufs[slot, 1, sli] = cmbufs[slot, 17, sli].astype(
                        jnp.int32)
                pltpu.async_copy(hw_hbm.at[ibufs.at[slot, 0]],
                                 gbufs.at[slot], sems.at[slot])

            load_chunk(0, 0)

            def pair(jj, acc8):
                for b in range(2):
                    j = 2 * jj + b

                    @pl.when(j + 1 < SYM_CH)
                    def _prefetch():
                        load_chunk(j + 1, 1 - b)

                    pltpu.make_async_copy(hw_hbm.at[ibufs.at[b, 0]],
                                          gbufs.at[b], sems.at[b]).wait()
                    gbuf = gbufs.at[b]
                    cmbuf = cmbufs.at[b]

                    def upd(r, a8):
                        for q in range(8):
                            e = 8 * r + q
                            v = cmbuf[r, pl.ds(16 * q, 16)]
                            nsp = lax.abs(v)
                            ksp = jnp.where(v < 0.0, one, zero)
                            a8 = tuple(
                                gbuf[e, pl.ds(16 * k, 16)] * nsp
                                + ksp * a8[k]
                                for k in range(8))
                            for k in range(8):
                                gbuf[e, pl.ds(16 * k, 16)] = a8[k]
                        return a8

                    acc8 = lax.fori_loop(0, C // 8, upd, acc8)
                    pltpu.sync_copy(gbufs.at[b], acc.at[ibufs.at[b, 1]],
                                    add=True)
                return acc8

            lax.fori_loop(0, SYM_CH // 2, pair, zero8)

        plsc.subcore_barrier()
        pltpu.sync_copy(acc.at[pl.ds(s * RPT, RPT)],
                        out_hbm.at[c, pl.ds(s * RPT, RPT)])

    @functools.partial(
        pl.kernel,
        out_type=jax.ShapeDtypeStruct((NW2 * CAP2, HID), jnp.float32),
        mesh=mesh,
        scratch_types=[
            pltpu.VMEM((C,), jnp.int32),
            pltpu.VMEM((C, HID), jnp.float32),
            pltpu.SemaphoreType.DMA,
        ],
    )
    def sym_gather(tab_hbm, idx_hbm, out_hbm, ibuf, gbuf, sem):
        c = lax.axis_index("c")
        s = lax.axis_index("s")
        wid = s * NC + c

        @pl.when(wid < NW2)
        def _active():
            def chunk(j, carry):
                pltpu.sync_copy(idx_hbm.at[wid, j], ibuf)
                pltpu.async_copy(tab_hbm.at[ibuf], gbuf, sem).wait()
                base = wid * CAP2 + j * C
                pltpu.sync_copy(gbuf, out_hbm.at[pl.ds(base, C)])
                return carry

            lax.fori_loop(0, SYM_CH, chunk, 0)

    return edge_scatter, pair_gather, sym_scatter, sym_gather


RB = 1000
NG = N // RB
PB = 1024
PG = P // PB

_CT1 = (((1,), (1,)), ((), ()))
_CT0 = (((1,), (0,)), ((), ()))


def _enc_body(x_ref, w_ref, h_ref):
    h_ref[...] = jnp.maximum(
        lax.dot_general(x_ref[...], w_ref[...], _CT1,
                        preferred_element_type=jnp.float32), 0.0)


_encode = pl.pallas_call(
    _enc_body,
    grid=(NG,),
    in_specs=[pl.BlockSpec((RB, IN_DIM), lambda i: (i, 0)),
              pl.BlockSpec((HID, IN_DIM), lambda i: (0, 0))],
    out_specs=pl.BlockSpec((RB, HID), lambda i: (i, 0)),
    out_shape=jax.ShapeDtypeStruct((N, HID), jnp.float32),
)


def _dis_body(dp_ref, dis_ref):
    deg = dp_ref[0, :, 0:1] + dp_ref[1, :, 0:1] + 1.0
    dis_ref[...] = jnp.broadcast_to(lax.rsqrt(deg), (NR, HID))


_dis_k = pl.pallas_call(
    _dis_body,
    in_specs=[pl.BlockSpec((NC, NR, HID), lambda: (0, 0, 0))],
    out_specs=pl.BlockSpec((NR, HID), lambda: (0, 0)),
    out_shape=jax.ShapeDtypeStruct((NR, HID), jnp.float32),
)


def _msg_body(h_ref, oa_ref, ws_ref, anti_ref, hw_ref):
    h = h_ref[...]
    anti_ref[...] = -jnp.maximum(
        lax.dot_general(h, oa_ref[...], _CT0,
                        preferred_element_type=jnp.float32), 0.0)
    hw_ref[...] = lax.dot_general(h, ws_ref[...], _CT0,
                                  preferred_element_type=jnp.float32)


_msg = pl.pallas_call(
    _msg_body,
    grid=(NG,),
    in_specs=[pl.BlockSpec((RB, HID), lambda i: (i, 0)),
              pl.BlockSpec((HID, HID), lambda i: (0, 0)),
              pl.BlockSpec((HID, HID), lambda i: (0, 0))],
    out_specs=[pl.BlockSpec((RB, HID), lambda i: (i, 0)),
               pl.BlockSpec((RB, HID), lambda i: (i, 0))],
    out_shape=[jax.ShapeDtypeStruct((N, HID), jnp.float32),
               jax.ShapeDtypeStruct((N, HID), jnp.float32)],
)


def _upd_body(anti_ref, p_ref, h_ref, hn_ref, dh_ref):
    sym = p_ref[0] + p_ref[1]
    dh = jnp.maximum(jnp.tanh(anti_ref[...] + sym), 0.0)
    dh_ref[...] = dh
    hn_ref[...] = h_ref[...] + dh


_upd = pl.pallas_call(
    _upd_body,
    grid=(NG,),
    in_specs=[pl.BlockSpec((RB, HID), lambda i: (i, 0)),
              pl.BlockSpec((NC, RB, HID), lambda i: (0, i, 0)),
              pl.BlockSpec((RB, HID), lambda i: (i, 0))],
    out_specs=[pl.BlockSpec((RB, HID), lambda i: (i, 0)),
               pl.BlockSpec((RB, HID), lambda i: (i, 0))],
    out_shape=[jax.ShapeDtypeStruct((N, HID), jnp.float32),
               jax.ShapeDtypeStruct((N, HID), jnp.float32)],
)


def _gate_body(hs_ref, hd_ref, dhs_ref, dhd_ref, rs_ref, rd_ref, ts_ref,
               w1_ref, w2_ref, g_ref,
               hs_o, hd_o, rs_o, rd_o, ts_o):
    huv = jnp.concatenate([hs_ref[...], hd_ref[...]], axis=1)
    mm = jnp.maximum(
        lax.dot_general(huv, w1_ref[...], _CT1,
                        preferred_element_type=jnp.float32), 0.0)
    a = lax.dot_general(mm, w2_ref[...], _CT1,
                        preferred_element_type=jnp.float32) + g_ref[...]
    tau = jnp.where(a[:, 0:1] >= a[:, 1:2], 1.0, 0.0).astype(jnp.float32)
    dhs = dhs_ref[...]
    dhd = dhd_ref[...]
    hs_o[...] = hs_ref[...] + dhs
    hd_o[...] = hd_ref[...] + dhd
    rs_o[...] = rs_ref[...] + tau * dhs
    rd_o[...] = rd_ref[...] + tau * dhd
    ts_o[...] = ts_ref[...] + tau


_gate = pl.pallas_call(
    _gate_body,
    grid=(PG,),
    in_specs=[pl.BlockSpec((PB, HID), lambda i: (i, 0)),
              pl.BlockSpec((PB, HID), lambda i: (i, 0)),
              pl.BlockSpec((PB, HID), lambda i: (i, 0)),
              pl.BlockSpec((PB, HID), lambda i: (i, 0)),
              pl.BlockSpec((PB, HID), lambda i: (i, 0)),
              pl.BlockSpec((PB, HID), lambda i: (i, 0)),
              pl.BlockSpec((PB, 1), lambda i: (i, 0)),
              pl.BlockSpec((2 * HID, 2 * HID), lambda i: (0, 0)),
              pl.BlockSpec((2, 2 * HID), lambda i: (0, 0)),
              pl.BlockSpec((PB, 2), lambda i: (i, 0))],
    out_specs=[pl.BlockSpec((PB, HID), lambda i: (i, 0)),
               pl.BlockSpec((PB, HID), lambda i: (i, 0)),
               pl.BlockSpec((PB, HID), lambda i: (i, 0)),
               pl.BlockSpec((PB, HID), lambda i: (i, 0)),
               pl.BlockSpec((PB, 1), lambda i: (i, 0))],
    out_shape=[jax.ShapeDtypeStruct((P, HID), jnp.float32),
               jax.ShapeDtypeStruct((P, HID), jnp.float32),
               jax.ShapeDtypeStruct((P, HID), jnp.float32),
               jax.ShapeDtypeStruct((P, HID), jnp.float32),
               jax.ShapeDtypeStruct((P, 1), jnp.float32)],
)


def _score_body(rs_ref, rd_ref, p1_ref, p2_ref, sc_o):
    r = jnp.concatenate([rs_ref[...], rd_ref[...]], axis=1)
    q = jnp.maximum(
        lax.dot_general(r, p1_ref[...], _CT1,
                        preferred_element_type=jnp.float32), 0.0)
    sc_o[...] = lax.dot_general(q, p2_ref[...], _CT1,
                                preferred_element_type=jnp.float32)


_score = pl.pallas_call(
    _score_body,
    grid=(PG,),
    in_specs=[pl.BlockSpec((PB, HID), lambda i: (i, 0)),
              pl.BlockSpec((PB, HID), lambda i: (i, 0)),
              pl.BlockSpec((HID, 2 * HID), lambda i: (0, 0)),
              pl.BlockSpec((1, HID), lambda i: (0, 0))],
    out_specs=pl.BlockSpec((PB, 1), lambda i: (i, 0)),
    out_shape=jax.ShapeDtypeStruct((P, 1), jnp.float32),
)


def kernel(x, edge_index, pairs, W_enc, Omega, W_s_raw, W_fc1, W_fc2, W_fv,
           W_p1, W_p2):
    del W_fv
    Omega_as = Omega - Omega.T
    W_s = (W_s_raw + W_s_raw.T) / 2.0
    idt = edge_index.dtype

    pad = EP - E
    rows_p = jnp.concatenate(
        [edge_index[0], jnp.zeros((pad,), idt)]).reshape(NW, EPW_CH, C)
    cols_p = jnp.concatenate(
        [edge_index[1], jnp.full((pad,), DUMP, idt)]).reshape(NW, EPW_CH, C)
    idx_pairs = jnp.concatenate(
        [pairs[:, 0], pairs[:, 1]]).reshape(NW, GPW_CH, C)

    sl = jnp.arange(N, dtype=idt)
    rows_all = jnp.concatenate([edge_index[0], sl])
    cols_all = jnp.concatenate([edge_index[1], sl])
    order = jnp.argsort(cols_all, stable=True)
    sr = rows_all[order]
    sc_ = cols_all[order]

    gpos = (jnp.arange(NW2) * XCS)[:, None] + jnp.arange(CAP2)[None, :]
    local = jnp.broadcast_to(jnp.arange(CAP2)[None, :], (NW2, CAP2))
    valid = (gpos < M_UPD) & (local < XCS)
    gclip = jnp.minimum(gpos, M_UPD - 1)
    rows_m = jnp.where(valid, sr[gclip], 0).astype(jnp.int32)
    cols_m = jnp.where(valid, sc_[gclip], DUMP).astype(jnp.int32)
    prevc = jnp.concatenate(
        [jnp.full((NW2, 1), -1, jnp.int32), cols_m[:, :-1]], axis=1)
    keep_m = ((cols_m == prevc) & valid).astype(jnp.float32)
    nxtc = jnp.concatenate(
        [cols_m[:, 1:], jnp.full((NW2, 1), -2, jnp.int32)], axis=1)
    fcol_m = jnp.where((cols_m != nxtc) & valid, cols_m, DUMP)
    rows_m3 = rows_m.reshape(NW2, SYM_CH, C)
    cols_m3 = cols_m.reshape(NW2, SYM_CH, C)
    keep_m3 = keep_m.reshape(NW2, SYM_CH, C)
    fcol_m3 = fcol_m.reshape(NW2, SYM_CH, C)

    zeros_nodes = jnp.zeros((NR, HID), jnp.float32)
    ones_nodes = jnp.ones((N, HID), jnp.float32)

    gs = [jax.random.gumbel(jax.random.fold_in(jax.random.key(42), l),
                            (P, 2), jnp.float32) for l in range(L)]

    edge_scatter, pair_gather, sym_scatter, sym_gather = _sc_kernels()

    H = _encode(x, W_enc)
    degp = edge_scatter(ones_nodes, rows_p, cols_p, zeros_nodes)
    dis128 = _dis_k(degp)
    d_a = sym_gather(dis128, rows_m3)
    d_b = sym_gather(dis128, cols_m3)
    norm_flat = jnp.where(valid.reshape(-1), d_a[:, 0] * d_b[:, 0], 0.0)
    signed = jnp.where(keep_m.reshape(-1) > 0, -norm_flat, norm_flat)
    nk_rows = jnp.broadcast_to(
        signed[:, None], (NW2 * CAP2, 16)).reshape(NW2, SYM_CH, 16, C)
    cmb = jnp.concatenate(
        [nk_rows,
         rows_m.reshape(NW2, SYM_CH, 1, C).astype(jnp.float32),
         fcol_m.reshape(NW2, SYM_CH, 1, C).astype(jnp.float32),
         jnp.zeros((NW2, SYM_CH, 6, C), jnp.float32)], axis=2)

    huv0 = pair_gather(H, idx_pairs)
    hs = huv0[:P]
    hd = huv0[P:]
    rs = hs
    rd = hd
    ts = jnp.zeros((P, 1), jnp.float32)

    for l in range(L):
        anti, HW = _msg(H, Omega_as, W_s)
        pparts = sym_scatter(HW, cmb, zeros_nodes)
        H, dH = _upd(anti, pparts, H)
        dhuv = pair_gather(dH, idx_pairs)
        hs, hd, rs, rd, ts = _gate(hs, hd, dhuv[:P], dhuv[P:], rs, rd, ts,
                                   W_fc1, W_fc2, gs[l])

    scores = _score(rs, rd, W_p1, W_p2)
    return (scores[:, 0], ts[:, 0])
